# Initial kernel scaffold; baseline (speedup 1.0000x reference)
import jax
import jax.numpy as jnp
from jax.experimental import pallas as pl


def kernel(pos_org, pos_shuffled, feat, feat_prev, W_proj, b_proj, ln_g, ln_b, W1, b1, W2, b2, W3, b3):
    raise NotImplementedError("write your pallas kernel here")


# speedup vs baseline: 2.8810x; 2.8810x over previous
import functools

import jax
import jax.numpy as jnp
from jax import lax
from jax.experimental import pallas as pl
from jax.experimental.pallas import tpu as pltpu
from jax.experimental.pallas import tpu_sc as plsc

_TQ = 256
_TN = 256
_SC_LANES = 16


def _argmin_body(n_keys, q_ref, k_ref, idx_ref):
    qx = q_ref[0, 0:1, :]
    qy = q_ref[0, 1:2, :]
    kx = k_ref[0, :, 0:1]
    ky = k_ref[0, :, 1:2]
    d = jnp.abs(kx - qx) + jnp.abs(ky - qy)
    dmin = jnp.min(d, axis=0, keepdims=True)
    ii = lax.broadcasted_iota(jnp.int32, d.shape, 0)
    sel = jnp.where(d == dmin, ii, n_keys)
    idx_ref[0, 0, :] = jnp.min(sel, axis=0)


def _nn_indices(pos_org, pos_shuffled):
    B, N, _ = pos_org.shape
    pos_org_t = jnp.transpose(pos_org, (0, 2, 1))
    grid = (B, N // _TQ)
    idx = pl.pallas_call(
        functools.partial(_argmin_body, N),
        grid=grid,
        in_specs=[
            pl.BlockSpec((1, 2, _TQ), lambda b, j: (b, 0, j)),
            pl.BlockSpec((1, N, 2), lambda b, j: (b, 0, 0)),
        ],
        out_specs=pl.BlockSpec((1, 1, _TQ), lambda b, j: (b, 0, j)),
        out_shape=jax.ShapeDtypeStruct((B, 1, N), jnp.int32),
        compiler_params=pltpu.CompilerParams(
            dimension_semantics=("parallel", "parallel"),
        ),
    )(pos_org_t, pos_shuffled)
    return idx.reshape(B, N)


def _sc_hist_body(n_bins, n_batches, idx_hbm, cnt_hbm, idx_v, cnt_v):
    wid = lax.axis_index("s") * 2 + lax.axis_index("c")

    @pl.when(wid < n_batches)
    def _():
        pltpu.sync_copy(idx_hbm.at[wid], idx_v)
        zeros = jnp.zeros((_SC_LANES,), jnp.float32)
        ones = jnp.ones((_SC_LANES,), jnp.float32)

        def zero_body(i, carry):
            cnt_v[pl.ds(i * _SC_LANES, _SC_LANES)] = zeros
            return carry

        lax.fori_loop(0, n_bins // _SC_LANES, zero_body, 0)

        def add_body(i, carry):
            iv = idx_v[pl.ds(i * _SC_LANES, _SC_LANES)]
            plsc.addupdate_scatter(cnt_v, [iv], ones)
            return carry

        lax.fori_loop(0, n_bins // _SC_LANES, add_body, 0)
        pltpu.sync_copy(cnt_v, cnt_hbm.at[wid])


def _index_histogram(idx):
    B, N = idx.shape
    mesh = plsc.VectorSubcoreMesh(core_axis_name="c", subcore_axis_name="s")
    hist = pl.kernel(
        functools.partial(_sc_hist_body, N, B),
        mesh=mesh,
        out_type=jax.ShapeDtypeStruct((B, N), jnp.float32),
        scratch_types=[
            pltpu.VMEM((N,), jnp.int32),
            pltpu.VMEM((N,), jnp.float32),
        ],
        compiler_params=pltpu.CompilerParams(needs_layout_passes=False),
    )
    return hist(idx)


def _proj_body(inv_rows, feat_ref, fprev_ref, w_ref, bp_ref, g_ref, b_ref,
               cnt_ref, acc_ref):
    j = pl.program_id(1)

    @pl.when(j == 0)
    def _():
        acc_ref[...] = jnp.zeros_like(acc_ref)

    x = feat_ref[0]
    proj = jnp.dot(x, w_ref[...], preferred_element_type=jnp.float32)
    proj = proj + bp_ref[0:1, :]
    mu = jnp.mean(proj, axis=1, keepdims=True)
    var = jnp.mean((proj - mu) ** 2, axis=1, keepdims=True)
    ln = (proj - mu) / jnp.sqrt(var + 1e-5) * g_ref[0:1, :] + b_ref[0:1, :]
    c = cnt_ref[0, :, :]
    wsum = jnp.dot(c, ln, preferred_element_type=jnp.float32)
    psum = jnp.sum(fprev_ref[0], axis=0, keepdims=True)
    acc_ref[0, :, :] += jnp.concatenate([wsum, psum], axis=1) * inv_rows


def _pooled(feat, feat_prev, counts, W_proj, b_proj, ln_g, ln_b):
    B, N, C = feat.shape
    grid = (B, N // _TN)
    acc = pl.pallas_call(
        functools.partial(_proj_body, 1.0 / N),
        grid=grid,
        in_specs=[
            pl.BlockSpec((1, _TN, C), lambda b, j: (b, j, 0)),
            pl.BlockSpec((1, _TN, C), lambda b, j: (b, j, 0)),
            pl.BlockSpec((C, C), lambda b, j: (0, 0)),
            pl.BlockSpec((1, C), lambda b, j: (0, 0)),
            pl.BlockSpec((1, C), lambda b, j: (0, 0)),
            pl.BlockSpec((1, C), lambda b, j: (0, 0)),
            pl.BlockSpec((1, 1, _TN), lambda b, j: (b, 0, j)),
        ],
        out_specs=pl.BlockSpec((1, 1, 2 * C), lambda b, j: (b, 0, 0)),
        out_shape=jax.ShapeDtypeStruct((B, 1, 2 * C), jnp.float32),
        compiler_params=pltpu.CompilerParams(
            dimension_semantics=("parallel", "arbitrary"),
        ),
    )(feat, feat_prev, W_proj, b_proj.reshape(1, C), ln_g.reshape(1, C),
      ln_b.reshape(1, C), counts.reshape(B, 1, N))
    return acc.reshape(B, 2 * C)


def _mlp_body(C, s_ref, w1_ref, b1_ref, w2_ref, b2_ref, w3_ref, b3_ref, o_ref):
    m = s_ref[...]
    fused_mean = m[:, 0:C] + m[:, C:2 * C]
    pooled = jnp.concatenate([fused_mean, m[:, C:2 * C]], axis=1)
    h = jnp.dot(pooled, w1_ref[...], preferred_element_type=jnp.float32)
    h = jnp.maximum(h + b1_ref[0:1, :], 0.0)
    h = jnp.dot(h, w2_ref[...], preferred_element_type=jnp.float32)
    h = jnp.maximum(h + b2_ref[0:1, :], 0.0)
    o = jnp.dot(h, w3_ref[...], preferred_element_type=jnp.float32)
    o_ref[...] = o + b3_ref[0:1, :]


def _mlp_head(pooled_means, W1, b1, W2, b2, W3, b3):
    B = pooled_means.shape[0]
    C = W2.shape[0]
    NC = W3.shape[1]
    return pl.pallas_call(
        functools.partial(_mlp_body, C),
        out_shape=jax.ShapeDtypeStruct((B, NC), jnp.float32),
    )(pooled_means, W1, b1.reshape(1, C), W2, b2.reshape(1, C), W3,
      b3.reshape(1, NC))


def kernel(pos_org, pos_shuffled, feat, feat_prev, W_proj, b_proj, ln_g, ln_b,
           W1, b1, W2, b2, W3, b3):
    idx = _nn_indices(pos_org, pos_shuffled)
    counts = _index_histogram(idx)
    pooled = _pooled(feat, feat_prev, counts, W_proj, b_proj, ln_g, ln_b)
    return _mlp_head(pooled, W1, b1, W2, b2, W3, b3)
